# baseline (speedup 1.0000x reference)
import functools

import jax
import jax.numpy as jnp
from jax import lax
from jax.experimental import pallas as pl
from jax.experimental.pallas import tpu as pltpu
from jax.experimental.pallas import tpu_sc as plsc

K = 8192
D = 32
TN = 1024
KC = 4096


def _argmin_body(x_ref, et_ref, zs_ref, es_ref, it_ref, idx_ref, loss_ref):
    x = x_ref[...].astype(jnp.float32)
    zs = zs_ref[...]
    acc_v = jnp.full((TN, 1), jnp.inf, jnp.float32)
    acc_i = jnp.zeros((TN, 1), jnp.int32)
    acc_clean = jnp.full((TN, 1), jnp.inf, jnp.float32)
    for c in range(K // KC):
        et = et_ref[:, c * KC:(c + 1) * KC]
        es = es_ref[:, c * KC:(c + 1) * KC]
        mm2 = lax.dot_general(x, et, (((1,), (0,)), ((), ())),
                              preferred_element_type=jnp.float32)
        dist = (zs + es) - mm2
        vmin = jnp.min(dist, axis=1, keepdims=True)
        kidx = it_ref[:, c * KC:(c + 1) * KC]
        imin_f = jnp.min(jnp.where(dist == vmin, kidx, jnp.float32(K)),
                         axis=1, keepdims=True)
        imin = imin_f.astype(jnp.int32)
        keep = (acc_v < vmin) | ((acc_v == vmin) & (acc_i < imin))
        acc_i = jnp.where(keep, acc_i, imin)
        acc_clean = jnp.where(keep, acc_clean, vmin)
        acc_v = jnp.where(keep, acc_v, vmin).astype(jnp.bfloat16).astype(
            jnp.float32)
    best_idx, best_val = acc_i, acc_clean
    idx_ref[0, 0, :] = best_idx[:, 0]
    i = pl.program_id(0)

    @pl.when(i == 0)
    def _init():
        loss_ref[...] = jnp.zeros((1, 1), jnp.float32)

    loss_ref[...] += jnp.sum(best_val, keepdims=True)


def _distance_argmin(flat_z, emb_t, zs, es):
    nb = flat_z.shape[0] // TN
    iota_row = lax.broadcasted_iota(jnp.float32, (1, K), 1)
    return pl.pallas_call(
        _argmin_body,
        grid=(nb,),
        in_specs=[
            pl.BlockSpec((TN, D), lambda i: (i, 0)),
            pl.BlockSpec((D, K), lambda i: (0, 0)),
            pl.BlockSpec((TN, 1), lambda i: (i, 0)),
            pl.BlockSpec((1, K), lambda i: (0, 0)),
            pl.BlockSpec((1, K), lambda i: (0, 0)),
        ],
        out_specs=[
            pl.BlockSpec((1, 1, TN), lambda i: (i, 0, 0)),
            pl.BlockSpec((1, 1), lambda i: (0, 0)),
        ],
        out_shape=[
            jax.ShapeDtypeStruct((nb, 1, TN), jnp.int32),
            jax.ShapeDtypeStruct((1, 1), jnp.float32),
        ],
    )(flat_z, emb_t, zs, es, iota_row)


def _st_body(z_ref, q_ref, out_ref):
    q = q_ref[...]
    qt = jnp.swapaxes(q, 0, 1)
    zb = z_ref[0]
    out_ref[0] = zb + (qt - zb)


def _straight_through(z3, quant_flat, b, hw):
    return pl.pallas_call(
        _st_body,
        grid=(b,),
        in_specs=[
            pl.BlockSpec((1, D, hw), lambda i: (i, 0, 0)),
            pl.BlockSpec((hw, D), lambda i: (i, 0)),
        ],
        out_specs=pl.BlockSpec((1, D, hw), lambda i: (i, 0, 0)),
        out_shape=jax.ShapeDtypeStruct((b, D, hw), jnp.float32),
    )(z3, quant_flat)


def _sc_gather(table, idx, n):
    info = plsc.get_sparse_core_info()
    nw = info.num_cores * info.num_subcores
    b_per_w = n // nw
    mesh = plsc.VectorSubcoreMesh(core_axis_name="c", subcore_axis_name="s")

    @functools.partial(
        pl.kernel, mesh=mesh,
        compiler_params=pltpu.CompilerParams(use_tc_tiling_on_sc=False),
        out_type=jax.ShapeDtypeStruct((n, D), jnp.float32),
        scratch_types=[
            pltpu.VMEM((b_per_w,), jnp.int32),
            pltpu.VMEM((b_per_w, D), jnp.float32),
            pltpu.SemaphoreType.DMA,
        ],
    )
    def gather_kernel(table_hbm, idx_hbm, out_hbm, idx_v, rows_v, sem):
        wid = lax.axis_index("s") * info.num_cores + lax.axis_index("c")
        base = wid * b_per_w
        pltpu.sync_copy(idx_hbm.at[pl.ds(base, b_per_w)], idx_v)
        pltpu.async_copy(table_hbm.at[idx_v], rows_v, sem).wait()
        pltpu.sync_copy(rows_v, out_hbm.at[pl.ds(base, b_per_w)])

    return gather_kernel(table, idx)


def kernel(z, emb_weight):
    b, c, h, w = z.shape
    n = b * h * w
    zp = jnp.transpose(z, (0, 2, 3, 1))
    flat_z = zp.reshape(n, D)
    zs = jnp.sum(flat_z ** 2, axis=1, keepdims=True)
    es = jnp.sum(emb_weight ** 2, axis=1).reshape(1, K)
    emb_t = emb_weight.T
    flat_zb2 = flat_z.astype(jnp.bfloat16) * jnp.bfloat16(2)

    idx_blocks, loss_sum = _distance_argmin(flat_zb2, emb_t, zs, es)
    idx = idx_blocks.reshape(n)

    quant_flat = _sc_gather(emb_weight, idx, n)

    mse = loss_sum[0, 0] / jnp.float32(n * D)
    loss = mse + jnp.float32(0.25) * mse

    z3 = z.reshape(b, c, h * w)
    quantized_st = _straight_through(z3, quant_flat, b, h * w).reshape(z.shape)
    return quantized_st, loss


# speedup vs baseline: 1.0790x; 1.0790x over previous
import functools

import jax
import jax.numpy as jnp
from jax import lax
from jax.experimental import pallas as pl
from jax.experimental.pallas import tpu as pltpu
from jax.experimental.pallas import tpu_sc as plsc

K = 8192
D = 32
TN = 1024
KC = 4096


def _argmin_body(x_ref, et_ref, zs_ref, es_ref, it_ref, idx_ref, loss_ref):
    x = x_ref[...].astype(jnp.float32)
    zs = zs_ref[...]
    acc_v = jnp.full((TN, 1), jnp.inf, jnp.float32)
    acc_i = jnp.zeros((TN, 1), jnp.int32)
    acc_clean = jnp.full((TN, 1), jnp.inf, jnp.float32)
    for c in range(K // KC):
        et = et_ref[:, c * KC:(c + 1) * KC]
        es = es_ref[:, c * KC:(c + 1) * KC]
        mm2 = lax.dot_general(x, et, (((1,), (0,)), ((), ())),
                              preferred_element_type=jnp.float32)
        dist = (zs + es) - mm2
        vmin = jnp.min(dist, axis=1, keepdims=True)
        imin = jnp.argmin(dist, axis=1).astype(jnp.int32).reshape(TN, 1) + (
            c * KC)
        keep = (acc_v < vmin) | ((acc_v == vmin) & (acc_i < imin))
        acc_i = jnp.where(keep, acc_i, imin)
        acc_clean = jnp.where(keep, acc_clean, vmin)
        acc_v = jnp.where(keep, acc_v, vmin).astype(jnp.bfloat16).astype(
            jnp.float32)
    best_idx, best_val = acc_i, acc_clean
    idx_ref[0, 0, :] = best_idx[:, 0]
    i = pl.program_id(0)

    @pl.when(i == 0)
    def _init():
        loss_ref[...] = jnp.zeros((1, 1), jnp.float32)

    loss_ref[...] += jnp.sum(best_val, keepdims=True)


def _distance_argmin(flat_z, emb_t, zs, es):
    nb = flat_z.shape[0] // TN
    iota_row = lax.broadcasted_iota(jnp.float32, (1, K), 1)
    return pl.pallas_call(
        _argmin_body,
        grid=(nb,),
        in_specs=[
            pl.BlockSpec((TN, D), lambda i: (i, 0)),
            pl.BlockSpec((D, K), lambda i: (0, 0)),
            pl.BlockSpec((TN, 1), lambda i: (i, 0)),
            pl.BlockSpec((1, K), lambda i: (0, 0)),
            pl.BlockSpec((1, K), lambda i: (0, 0)),
        ],
        out_specs=[
            pl.BlockSpec((1, 1, TN), lambda i: (i, 0, 0)),
            pl.BlockSpec((1, 1), lambda i: (0, 0)),
        ],
        out_shape=[
            jax.ShapeDtypeStruct((nb, 1, TN), jnp.int32),
            jax.ShapeDtypeStruct((1, 1), jnp.float32),
        ],
    )(flat_z, emb_t, zs, es, iota_row)


def _st_body(z_ref, q_ref, out_ref):
    q = q_ref[...]
    qt = jnp.swapaxes(q, 0, 1)
    zb = z_ref[0]
    out_ref[0] = zb + (qt - zb)


def _straight_through(z3, quant_flat, b, hw):
    return pl.pallas_call(
        _st_body,
        grid=(b,),
        in_specs=[
            pl.BlockSpec((1, D, hw), lambda i: (i, 0, 0)),
            pl.BlockSpec((hw, D), lambda i: (i, 0)),
        ],
        out_specs=pl.BlockSpec((1, D, hw), lambda i: (i, 0, 0)),
        out_shape=jax.ShapeDtypeStruct((b, D, hw), jnp.float32),
    )(z3, quant_flat)


def _sc_gather(table, idx, n):
    info = plsc.get_sparse_core_info()
    nw = info.num_cores * info.num_subcores
    b_per_w = n // nw
    mesh = plsc.VectorSubcoreMesh(core_axis_name="c", subcore_axis_name="s")

    @functools.partial(
        pl.kernel, mesh=mesh,
        compiler_params=pltpu.CompilerParams(use_tc_tiling_on_sc=False),
        out_type=jax.ShapeDtypeStruct((n, D), jnp.float32),
        scratch_types=[
            pltpu.VMEM((b_per_w,), jnp.int32),
            pltpu.VMEM((b_per_w, D), jnp.float32),
            pltpu.SemaphoreType.DMA,
        ],
    )
    def gather_kernel(table_hbm, idx_hbm, out_hbm, idx_v, rows_v, sem):
        wid = lax.axis_index("s") * info.num_cores + lax.axis_index("c")
        base = wid * b_per_w
        pltpu.sync_copy(idx_hbm.at[pl.ds(base, b_per_w)], idx_v)
        pltpu.async_copy(table_hbm.at[idx_v], rows_v, sem).wait()
        pltpu.sync_copy(rows_v, out_hbm.at[pl.ds(base, b_per_w)])

    return gather_kernel(table, idx)


def kernel(z, emb_weight):
    b, c, h, w = z.shape
    n = b * h * w
    zp = jnp.transpose(z, (0, 2, 3, 1))
    flat_z = zp.reshape(n, D)
    zs = jnp.sum(flat_z ** 2, axis=1, keepdims=True)
    es = jnp.sum(emb_weight ** 2, axis=1).reshape(1, K)
    emb_t = emb_weight.T
    flat_zb2 = flat_z.astype(jnp.bfloat16) * jnp.bfloat16(2)

    idx_blocks, loss_sum = _distance_argmin(flat_zb2, emb_t, zs, es)
    idx = idx_blocks.reshape(n)

    quant_flat = _sc_gather(emb_weight, idx, n)

    mse = loss_sum[0, 0] / jnp.float32(n * D)
    loss = mse + jnp.float32(0.25) * mse

    quantized = jnp.transpose(quant_flat.reshape(b, h, w, c), (0, 3, 1, 2))
    quantized_st = z + (quantized - z)
    return quantized_st, loss
